# baseline (speedup 1.0000x reference)
import functools

import jax
import jax.numpy as jnp
from jax.experimental import pallas as pl
from jax.experimental.pallas import tpu as pltpu

EMBED_DIM = 768
NUM_EXPERTS = 8
TOP_K = 2
D_FF = 3072
D_FF_SH = 1536
SEQ = 2048

TOK_TILE = 256
FF_TILE = 768
N_TOK_TILES = SEQ // TOK_TILE
N_FF = D_FF // FF_TILE
N_FF_SH = D_FF_SH // FF_TILE


def _router_body(x_ref, gw_ref, wdense_ref):
    x = x_ref[...]
    logits = jax.lax.dot_general(
        x, gw_ref[...], (((1,), (1,)), ((), ())),
        preferred_element_type=jnp.float32)
    m = jnp.max(logits, axis=-1, keepdims=True)
    e = jnp.exp(logits - m)
    scores = e / jnp.sum(e, axis=-1, keepdims=True)
    iota = jax.lax.broadcasted_iota(jnp.int32, scores.shape, 1)
    big = jnp.int32(NUM_EXPERTS)
    m1 = jnp.max(scores, axis=-1, keepdims=True)
    i1 = jnp.min(jnp.where(scores == m1, iota, big), axis=-1, keepdims=True)
    excl = iota == i1
    neg = jnp.float32(-jnp.inf)
    masked = jnp.where(excl, neg, scores)
    m2 = jnp.max(masked, axis=-1, keepdims=True)
    i2 = jnp.min(jnp.where((scores == m2) & (~excl), iota, big),
                 axis=-1, keepdims=True)
    keep = (iota == i1) | (iota == i2)
    wdense_ref[...] = jnp.where(keep, scores, 0.0)


def _moe_body(x_ref, wg_ref, wu_ref, wd_ref, wdense_ref, out_ref):
    e = pl.program_id(1)
    f = pl.program_id(2)

    @pl.when((e == 0) & (f == 0))
    def _():
        out_ref[...] = jnp.zeros_like(out_ref)

    x = x_ref[...]
    g = jax.lax.dot_general(x, wg_ref[0], (((1,), (1,)), ((), ())),
                            preferred_element_type=jnp.float32)
    u = jax.lax.dot_general(x, wu_ref[0], (((1,), (1,)), ((), ())),
                            preferred_element_type=jnp.float32)
    h = (g * jax.lax.logistic(g) * u).astype(jnp.bfloat16)
    o = jax.lax.dot_general(h, wd_ref[0], (((1,), (1,)), ((), ())),
                            preferred_element_type=jnp.float32)
    iota = jax.lax.broadcasted_iota(jnp.int32, wdense_ref.shape, 1)
    w_col = jnp.sum(jnp.where(iota == e, wdense_ref[...], 0.0), axis=1,
                    keepdims=True)
    out_ref[...] += o * w_col


def _shared_body(x_ref, wg_ref, wu_ref, wd_ref, y_ref, out_ref):
    f = pl.program_id(1)

    @pl.when(f == 0)
    def _():
        out_ref[...] = y_ref[...]

    x = x_ref[...]
    g = jax.lax.dot_general(x, wg_ref[...], (((1,), (1,)), ((), ())),
                            preferred_element_type=jnp.float32)
    u = jax.lax.dot_general(x, wu_ref[...], (((1,), (1,)), ((), ())),
                            preferred_element_type=jnp.float32)
    h = (g * jax.lax.logistic(g) * u).astype(jnp.bfloat16)
    o = jax.lax.dot_general(h, wd_ref[...], (((1,), (1,)), ((), ())),
                            preferred_element_type=jnp.float32)
    out_ref[...] += o


@jax.jit
def kernel(hidden_states, gate_w, Wg, Wu, Wd, Wg_sh, Wu_sh, Wd_sh):
    b, s, d = hidden_states.shape
    x = hidden_states.reshape(s, d)
    xb = x.astype(jnp.bfloat16)

    wdense = pl.pallas_call(
        _router_body,
        grid=(N_TOK_TILES,),
        in_specs=[
            pl.BlockSpec((TOK_TILE, d), lambda t: (t, 0)),
            pl.BlockSpec((NUM_EXPERTS, d), lambda t: (0, 0)),
        ],
        out_specs=pl.BlockSpec((TOK_TILE, NUM_EXPERTS), lambda t: (t, 0)),
        out_shape=jax.ShapeDtypeStruct((s, NUM_EXPERTS), jnp.float32),
    )(x, gate_w)

    y_moe = pl.pallas_call(
        _moe_body,
        grid=(N_TOK_TILES, NUM_EXPERTS, N_FF),
        in_specs=[
            pl.BlockSpec((TOK_TILE, d), lambda t, e, f: (t, 0)),
            pl.BlockSpec((1, FF_TILE, d), lambda t, e, f: (e, f, 0)),
            pl.BlockSpec((1, FF_TILE, d), lambda t, e, f: (e, f, 0)),
            pl.BlockSpec((1, d, FF_TILE), lambda t, e, f: (e, 0, f)),
            pl.BlockSpec((TOK_TILE, NUM_EXPERTS), lambda t, e, f: (t, 0)),
        ],
        out_specs=pl.BlockSpec((TOK_TILE, d), lambda t, e, f: (t, 0)),
        out_shape=jax.ShapeDtypeStruct((s, d), jnp.float32),
    )(xb, Wg.astype(jnp.bfloat16), Wu.astype(jnp.bfloat16),
      Wd.astype(jnp.bfloat16), wdense)

    y = pl.pallas_call(
        _shared_body,
        grid=(N_TOK_TILES, N_FF_SH),
        in_specs=[
            pl.BlockSpec((TOK_TILE, d), lambda t, f: (t, 0)),
            pl.BlockSpec((FF_TILE, d), lambda t, f: (f, 0)),
            pl.BlockSpec((FF_TILE, d), lambda t, f: (f, 0)),
            pl.BlockSpec((d, FF_TILE), lambda t, f: (0, f)),
            pl.BlockSpec((TOK_TILE, d), lambda t, f: (t, 0)),
        ],
        out_specs=pl.BlockSpec((TOK_TILE, d), lambda t, f: (t, 0)),
        out_shape=jax.ShapeDtypeStruct((s, d), jnp.float32),
    )(xb, Wg_sh.astype(jnp.bfloat16), Wu_sh.astype(jnp.bfloat16),
      Wd_sh.astype(jnp.bfloat16), y_moe)

    return y.reshape(b, s, d)


# speedup vs baseline: 2.6631x; 2.6631x over previous
import functools

import jax
import jax.numpy as jnp
from jax import lax
from jax.experimental import pallas as pl
from jax.experimental.pallas import tpu as pltpu
from jax.experimental.pallas import tpu_sc as plsc

EMBED_DIM = 768
NUM_EXPERTS = 8
D_FF = 3072
D_FF_SH = 1536
SEQ = 2048
NPAIR = 2 * SEQ
ROW_TILE = 512
NTILES = 16
NROWS = NTILES * ROW_TILE
FF_TILE = 1536
N_FF = D_FF // FF_TILE
N_FF_SH = D_FF_SH // FF_TILE
TOK_TILE = 256
N_TOK_TILES = SEQ // TOK_TILE
L = 16


def _router_body(x_ref, gw_ref, eid_ref, wk_ref, hist_ref):
    x = x_ref[...]
    logits = lax.dot_general(x, gw_ref[...], (((1,), (1,)), ((), ())),
                             preferred_element_type=jnp.float32)
    m = jnp.max(logits, axis=-1, keepdims=True)
    e = jnp.exp(logits - m)
    scores = e / jnp.sum(e, axis=-1, keepdims=True)
    iota = lax.broadcasted_iota(jnp.int32, scores.shape, 1)
    big = jnp.int32(NUM_EXPERTS)
    m1 = jnp.max(scores, axis=-1, keepdims=True)
    i1 = jnp.min(jnp.where(scores == m1, iota, big), axis=-1, keepdims=True)
    excl = iota == i1
    masked = jnp.where(excl, -jnp.inf, scores)
    m2 = jnp.max(masked, axis=-1, keepdims=True)
    i2 = jnp.min(jnp.where((scores == m2) & (~excl), iota, big),
                 axis=-1, keepdims=True)
    w1 = jnp.sum(jnp.where(iota == i1, scores, 0.0), axis=-1, keepdims=True)
    w2 = jnp.sum(jnp.where((iota == i2) & (~excl), scores, 0.0),
                 axis=-1, keepdims=True)
    eid_ref[0] = jnp.concatenate([i1, i2], axis=1)
    wk_ref[0] = jnp.concatenate([w1, w2], axis=1)
    lane = lax.broadcasted_iota(jnp.int32, (TOK_TILE, 128), 1)
    ind = ((i1 == lane).astype(jnp.int32) + (i2 == lane).astype(jnp.int32))
    row = lax.broadcasted_iota(jnp.int32, (TOK_TILE, 128), 0)
    for h in range(4):
        msk = (row >= h * 64) & (row < (h + 1) * 64)
        hist_ref[0, h, :] = jnp.sum(jnp.where(msk, ind, 0), axis=0)


NW_H = 16
NW_D = 32
PPW = NPAIR // NW_D
NV = PPW // L
GCH = 64
NCH = PPW // GCH


def _lane(v, e):
    iota = lax.iota(jnp.int32, L)
    return jnp.sum(jnp.where(iota == e, v, 0))


def _dispatch_body(eid_hbm, x_hbm, hist_hbm, xs_hbm, inv_hbm, meta_hbm,
                   eid_v, dst1_v, dst2_v, tok_v, tbl_v, meta_v, xbuf_v,
                   semg0, semg1, sems):
    wid = lax.axis_index("s") * 2 + lax.axis_index("c")
    base_p = wid * PPW
    iota = lax.iota(jnp.int32, L)

    pltpu.sync_copy(eid_hbm.at[pl.ds(base_p, PPW)], eid_v)
    pltpu.sync_copy(hist_hbm, tbl_v)

    tot = jnp.zeros((L,), jnp.int32)
    pre = jnp.zeros((L,), jnp.int32)
    for i in range(NW_D):
        row = tbl_v[i]
        tot = tot + row
        pre = pre + jnp.where(jnp.int32(i) < wid, row, 0)
    padded = (tot + ROW_TILE - 1) - ((tot + ROW_TILE - 1) &
                                     jnp.int32(ROW_TILE - 1))
    cum = plsc.cumsum(padded)
    base = cum - padded
    cumt = lax.shift_right_logical(cum, 9)

    @pl.when(wid == 0)
    def _():
        nact = _lane(cumt, NUM_EXPERTS - 1)
        for half in range(2):
            t = iota + half * L
            ex = jnp.zeros((L,), jnp.int32)
            for e in range(NUM_EXPERTS):
                ce = _lane(cumt, e)
                ex = ex + jnp.where(ce <= t, 1, 0)
            ex = jnp.minimum(ex, NUM_EXPERTS - 1)
            if half == 1:
                ex = jnp.where(iota == 8, nact, jnp.where(iota < 8, ex, 0))
            meta_v[pl.ds(half * L, L)] = ex
        pltpu.sync_copy(meta_v, meta_hbm)

    start = base + pre
    for v in range(NV):
        ids = eid_v[pl.ds(v * L, L)]
        dst = jnp.zeros((L,), jnp.int32)
        add = jnp.zeros((L,), jnp.int32)
        for e in range(NUM_EXPERTS):
            msk = ids == e
            rank = plsc.cumsum(jnp.where(msk, 1, 0))
            c = _lane(rank, L - 1)
            se = _lane(start, e)
            dst = jnp.where(msk, se + rank - 1, dst)
            add = add + jnp.where(iota == e, c, 0)
        start = start + add
        dst1_v[pl.ds(v * L, L)] = dst
        dst2_v[v // (GCH // L), pl.ds((v % (GCH // L)) * L, L)] = dst
        tok_v[pl.ds(v * L, L)] = lax.shift_right_logical(
            base_p + v * L + iota, 1)

    pltpu.sync_copy(dst1_v, inv_hbm.at[pl.ds(base_p, PPW)])

    semg = [semg0, semg1]
    gathers = [
        pltpu.async_copy(x_hbm.at[tok_v.at[pl.ds(g * GCH, GCH)]],
                         xbuf_v.at[g], semg[g])
        for g in range(NCH)
    ]
    scatters = []
    for g in range(NCH):
        gathers[g].wait()
        scatters.append(pltpu.async_copy(
            xbuf_v.at[g], xs_hbm.at[dst2_v.at[g]], sems))
    for c in scatters:
        c.wait()


def _gmm_body(meta_ref, xs_ref, wg_ref, wu_ref, wd_ref, out_ref):
    t = pl.program_id(0)
    f = pl.program_id(1)
    active = t < meta_ref[24]

    @pl.when(active)
    def _():
        @pl.when(f == 0)
        def _():
            out_ref[...] = jnp.zeros_like(out_ref)

        x = xs_ref[...].astype(jnp.bfloat16)
        g = lax.dot_general(x, wg_ref[0].astype(jnp.bfloat16),
                            (((1,), (1,)), ((), ())),
                            preferred_element_type=jnp.float32)
        u = lax.dot_general(x, wu_ref[0].astype(jnp.bfloat16),
                            (((1,), (1,)), ((), ())),
                            preferred_element_type=jnp.float32)
        h = (g * lax.logistic(g) * u).astype(jnp.bfloat16)
        o = lax.dot_general(h, wd_ref[0].astype(jnp.bfloat16),
                            (((1,), (1,)), ((), ())),
                            preferred_element_type=jnp.float32)
        out_ref[...] += o


NW_C = 32
PPW_C = NPAIR // NW_C
GCH_C = 64


def _combine_body(ys_hbm, inv_hbm, yu_hbm, inv_v, buf_v, sem):
    wid = lax.axis_index("s") * 2 + lax.axis_index("c")
    base = wid * PPW_C
    pltpu.sync_copy(inv_hbm.at[pl.ds(base, PPW_C)], inv_v)
    for g in range(PPW_C // GCH_C):
        pltpu.async_copy(ys_hbm.at[inv_v.at[pl.ds(g * GCH_C, GCH_C)]],
                         buf_v, sem).wait()
        pltpu.sync_copy(buf_v, yu_hbm.at[pl.ds(base + g * GCH_C, GCH_C)])


def _shared_body(x_ref, wg_ref, wu_ref, wd_ref, yu_ref, wk_ref, out_ref):
    f = pl.program_id(1)

    @pl.when(f == 0)
    def _():
        w = wk_ref[0]
        iota2 = lax.broadcasted_iota(jnp.int32, w.shape, 1)
        w1 = jnp.sum(jnp.where(iota2 == 0, w, 0.0), axis=1, keepdims=True)
        w2 = jnp.sum(jnp.where(iota2 == 1, w, 0.0), axis=1, keepdims=True)
        out_ref[...] = (yu_ref[:, :EMBED_DIM] * w1 +
                        yu_ref[:, EMBED_DIM:] * w2)

    x = x_ref[...]
    g = lax.dot_general(x, wg_ref[...].astype(jnp.bfloat16),
                        (((1,), (1,)), ((), ())),
                        preferred_element_type=jnp.float32)
    u = lax.dot_general(x, wu_ref[...].astype(jnp.bfloat16),
                        (((1,), (1,)), ((), ())),
                        preferred_element_type=jnp.float32)
    h = (g * lax.logistic(g) * u).astype(jnp.bfloat16)
    out_ref[...] += lax.dot_general(h, wd_ref[...].astype(jnp.bfloat16),
                                    (((1,), (1,)), ((), ())),
                                    preferred_element_type=jnp.float32)


def _dispatch_call(eid, x, hist):
    d = x.shape[1]
    mesh_d = plsc.VectorSubcoreMesh(core_axis_name="c", subcore_axis_name="s")
    return pl.kernel(
        _dispatch_body,
        out_type=[
            jax.ShapeDtypeStruct((NROWS, d), jnp.float32),
            jax.ShapeDtypeStruct((NPAIR,), jnp.int32),
            jax.ShapeDtypeStruct((2 * L,), jnp.int32),
        ],
        mesh=mesh_d,
        compiler_params=pltpu.CompilerParams(needs_layout_passes=False),
        scratch_types=[
            pltpu.VMEM((PPW,), jnp.int32),
            pltpu.VMEM((PPW,), jnp.int32),
            pltpu.VMEM((NCH, GCH), jnp.int32),
            pltpu.VMEM((PPW,), jnp.int32),
            pltpu.VMEM((NW_D, L), jnp.int32),
            pltpu.VMEM((2 * L,), jnp.int32),
            pltpu.VMEM((NCH, GCH, d), jnp.float32),
            pltpu.SemaphoreType.DMA,
            pltpu.SemaphoreType.DMA,
            pltpu.SemaphoreType.DMA,
        ],
    )(eid, x, hist)


def _combine_call(ys, inv):
    d = ys.shape[1]
    mesh_c = plsc.VectorSubcoreMesh(core_axis_name="c", subcore_axis_name="s")
    return pl.kernel(
        _combine_body,
        out_type=jax.ShapeDtypeStruct((NPAIR, d), jnp.float32),
        mesh=mesh_c,
        compiler_params=pltpu.CompilerParams(needs_layout_passes=False),
        scratch_types=[
            pltpu.VMEM((PPW_C,), jnp.int32),
            pltpu.VMEM((GCH_C, d), jnp.float32),
            pltpu.SemaphoreType.DMA,
        ],
    )(ys, inv)


def _router_call(x, gate_w, interpret=False):
    d = x.shape[1]
    return pl.pallas_call(
        _router_body,
        grid=(N_TOK_TILES,),
        in_specs=[
            pl.BlockSpec((TOK_TILE, d), lambda t: (t, 0)),
            pl.BlockSpec((NUM_EXPERTS, d), lambda t: (0, 0)),
        ],
        out_specs=[
            pl.BlockSpec((1, TOK_TILE, 2), lambda t: (t, 0, 0)),
            pl.BlockSpec((1, TOK_TILE, 2), lambda t: (t, 0, 0)),
            pl.BlockSpec((1, 4, 128), lambda t: (t, 0, 0)),
        ],
        out_shape=[
            jax.ShapeDtypeStruct((N_TOK_TILES, TOK_TILE, 2), jnp.int32),
            jax.ShapeDtypeStruct((N_TOK_TILES, TOK_TILE, 2), jnp.float32),
            jax.ShapeDtypeStruct((N_TOK_TILES, 4, 128), jnp.int32),
        ],
        interpret=interpret,
    )(x, gate_w)


def _gmm_call(meta, xs, Wg, Wu, Wd, interpret=False):
    d = xs.shape[1]

    def _tc(t, m):
        return jnp.minimum(t, m[24] - 1)

    def _fc(t, f, m):
        return jnp.where(t < m[24], (f + t) % N_FF,
                         (m[24] - 1 + N_FF - 1) % N_FF)

    return pl.pallas_call(
        _gmm_body,
        grid_spec=pltpu.PrefetchScalarGridSpec(
            num_scalar_prefetch=1,
            grid=(NTILES, N_FF),
            in_specs=[
                pl.BlockSpec((ROW_TILE, d), lambda t, f, m: (_tc(t, m), 0)),
                pl.BlockSpec((1, FF_TILE, d),
                             lambda t, f, m: (m[_tc(t, m)], _fc(t, f, m), 0)),
                pl.BlockSpec((1, FF_TILE, d),
                             lambda t, f, m: (m[_tc(t, m)], _fc(t, f, m), 0)),
                pl.BlockSpec((1, d, FF_TILE),
                             lambda t, f, m: (m[_tc(t, m)], 0, _fc(t, f, m))),
            ],
            out_specs=pl.BlockSpec((ROW_TILE, d),
                                   lambda t, f, m: (_tc(t, m), 0)),
        ),
        out_shape=jax.ShapeDtypeStruct((NROWS, d), jnp.float32),
        interpret=interpret,
    )(meta, xs, Wg, Wu, Wd)


def _shared_call(xb, Wg_sh, Wu_sh, Wd_sh, yu2, wk3, interpret=False):
    d = xb.shape[1]
    return pl.pallas_call(
        _shared_body,
        grid=(N_TOK_TILES, N_FF_SH),
        in_specs=[
            pl.BlockSpec((TOK_TILE, d), lambda t, f: (t, 0)),
            pl.BlockSpec((FF_TILE, d), lambda t, f: (f, 0)),
            pl.BlockSpec((FF_TILE, d), lambda t, f: (f, 0)),
            pl.BlockSpec((d, FF_TILE), lambda t, f: (0, f)),
            pl.BlockSpec((TOK_TILE, 2 * d), lambda t, f: (t, 0)),
            pl.BlockSpec((1, TOK_TILE, 2), lambda t, f: (t, 0, 0)),
        ],
        out_specs=pl.BlockSpec((TOK_TILE, d), lambda t, f: (t, 0)),
        out_shape=jax.ShapeDtypeStruct((xb.shape[0], d), jnp.float32),
        interpret=interpret,
    )(xb, Wg_sh, Wu_sh, Wd_sh, yu2, wk3)


@jax.jit
def kernel(hidden_states, gate_w, Wg, Wu, Wd, Wg_sh, Wu_sh, Wd_sh):
    b, s, d = hidden_states.shape
    x = hidden_states.reshape(s, d)
    xb = x.astype(jnp.bfloat16)

    eid3, wk3, hist3 = _router_call(x, gate_w)
    eid = eid3.reshape(NPAIR)
    hist = hist3[:, :, :L].reshape(NW_D, L)

    xs, inv, meta = _dispatch_call(eid, x, hist)
    ys = _gmm_call(meta, xs, Wg, Wu, Wd)
    yu = _combine_call(ys, inv)
    y = _shared_call(xb, Wg_sh, Wu_sh, Wd_sh, yu.reshape(SEQ, 2 * d), wk3)

    return y.reshape(b, s, d)


# speedup vs baseline: 2.6763x; 1.0049x over previous
import functools

import jax
import jax.numpy as jnp
from jax import lax
from jax.experimental import pallas as pl
from jax.experimental.pallas import tpu as pltpu
from jax.experimental.pallas import tpu_sc as plsc

EMBED_DIM = 768
NUM_EXPERTS = 8
D_FF = 3072
D_FF_SH = 1536
SEQ = 2048
NPAIR = 2 * SEQ
ROW_TILE = 512
NTILES = 16
NROWS = NTILES * ROW_TILE
FF_TILE = 1536
N_FF = D_FF // FF_TILE
N_FF_SH = D_FF_SH // FF_TILE
TOK_TILE = 256
N_TOK_TILES = SEQ // TOK_TILE
L = 16


def _router_body(x_ref, gw_ref, eid_ref, wk_ref, hist_ref, xb_ref):
    x = x_ref[...]
    xb_ref[...] = x.astype(jnp.bfloat16)
    logits = lax.dot_general(x, gw_ref[...], (((1,), (1,)), ((), ())),
                             preferred_element_type=jnp.float32)
    m = jnp.max(logits, axis=-1, keepdims=True)
    e = jnp.exp(logits - m)
    scores = e / jnp.sum(e, axis=-1, keepdims=True)
    iota = lax.broadcasted_iota(jnp.int32, scores.shape, 1)
    big = jnp.int32(NUM_EXPERTS)
    m1 = jnp.max(scores, axis=-1, keepdims=True)
    i1 = jnp.min(jnp.where(scores == m1, iota, big), axis=-1, keepdims=True)
    excl = iota == i1
    masked = jnp.where(excl, -jnp.inf, scores)
    m2 = jnp.max(masked, axis=-1, keepdims=True)
    i2 = jnp.min(jnp.where((scores == m2) & (~excl), iota, big),
                 axis=-1, keepdims=True)
    w1 = jnp.sum(jnp.where(iota == i1, scores, 0.0), axis=-1, keepdims=True)
    w2 = jnp.sum(jnp.where((iota == i2) & (~excl), scores, 0.0),
                 axis=-1, keepdims=True)
    eid_ref[0] = jnp.concatenate([i1, i2], axis=1)
    wk_ref[0] = jnp.concatenate([w1, w2], axis=1)
    lane = lax.broadcasted_iota(jnp.int32, (TOK_TILE, L), 1)
    ind = ((i1 == lane).astype(jnp.int32) + (i2 == lane).astype(jnp.int32))
    row = lax.broadcasted_iota(jnp.int32, (TOK_TILE, L), 0)
    for h in range(4):
        msk = (row >= h * 64) & (row < (h + 1) * 64)
        hist_ref[0, h, :] = jnp.sum(jnp.where(msk, ind, 0), axis=0)


NW_H = 16
NW_D = 32
PPW = NPAIR // NW_D
NV = PPW // L
GCH = 64
NCH = PPW // GCH


def _lane(v, e):
    iota = lax.iota(jnp.int32, L)
    return jnp.sum(jnp.where(iota == e, v, 0))


def _dispatch_body(eid_hbm, x_hbm, hist_hbm, xs_hbm, inv_hbm, meta_hbm,
                   eid_v, dst1_v, dst2_v, tok_v, tbl_v, meta_v, xbuf_v,
                   semg0, semg1, sems):
    wid = lax.axis_index("s") * 2 + lax.axis_index("c")
    base_p = wid * PPW
    iota = lax.iota(jnp.int32, L)

    pltpu.sync_copy(eid_hbm.at[pl.ds(base_p, PPW)], eid_v)
    pltpu.sync_copy(hist_hbm, tbl_v)

    tot = jnp.zeros((L,), jnp.int32)
    pre = jnp.zeros((L,), jnp.int32)
    for i in range(NW_D):
        row = tbl_v[i]
        tot = tot + row
        pre = pre + jnp.where(jnp.int32(i) < wid, row, 0)
    padded = (tot + ROW_TILE - 1) - ((tot + ROW_TILE - 1) &
                                     jnp.int32(ROW_TILE - 1))
    cum = plsc.cumsum(padded)
    base = cum - padded
    cumt = lax.shift_right_logical(cum, 9)

    @pl.when(wid == 0)
    def _():
        nact = _lane(cumt, NUM_EXPERTS - 1)
        for half in range(2):
            t = iota + half * L
            ex = jnp.zeros((L,), jnp.int32)
            for e in range(NUM_EXPERTS):
                ce = _lane(cumt, e)
                ex = ex + jnp.where(ce <= t, 1, 0)
            ex = jnp.minimum(ex, NUM_EXPERTS - 1)
            if half == 1:
                ex = jnp.where(iota == 8, nact, jnp.where(iota < 8, ex, 0))
            meta_v[pl.ds(half * L, L)] = ex
        pltpu.sync_copy(meta_v, meta_hbm)

    start = base + pre
    for v in range(NV):
        ids = eid_v[pl.ds(v * L, L)]
        dst = jnp.zeros((L,), jnp.int32)
        add = jnp.zeros((L,), jnp.int32)
        for e in range(NUM_EXPERTS):
            msk = ids == e
            rank = plsc.cumsum(jnp.where(msk, 1, 0))
            c = _lane(rank, L - 1)
            se = _lane(start, e)
            dst = jnp.where(msk, se + rank - 1, dst)
            add = add + jnp.where(iota == e, c, 0)
        start = start + add
        dst1_v[pl.ds(v * L, L)] = dst
        dst2_v[v // (GCH // L), pl.ds((v % (GCH // L)) * L, L)] = dst
        tok_v[pl.ds(v * L, L)] = lax.shift_right_logical(
            base_p + v * L + iota, 1)

    pltpu.sync_copy(dst1_v, inv_hbm.at[pl.ds(base_p, PPW)])

    semg = [semg0, semg1]
    gathers = [
        pltpu.async_copy(x_hbm.at[tok_v.at[pl.ds(g * GCH, GCH)]],
                         xbuf_v.at[g], semg[g])
        for g in range(NCH)
    ]
    scatters = []
    for g in range(NCH):
        gathers[g].wait()
        scatters.append(pltpu.async_copy(
            xbuf_v.at[g], xs_hbm.at[dst2_v.at[g]], sems))
    for c in scatters:
        c.wait()


def _gmm_body(meta_ref, xs_ref, wg_ref, wu_ref, wd_ref, out_ref):
    t = pl.program_id(0)
    f = pl.program_id(1)
    active = t < meta_ref[24]

    @pl.when(active)
    def _():
        @pl.when(f == 0)
        def _():
            out_ref[...] = jnp.zeros_like(out_ref)

        x = xs_ref[...].astype(jnp.bfloat16)
        g = lax.dot_general(x, wg_ref[0].astype(jnp.bfloat16),
                            (((1,), (1,)), ((), ())),
                            preferred_element_type=jnp.float32)
        u = lax.dot_general(x, wu_ref[0].astype(jnp.bfloat16),
                            (((1,), (1,)), ((), ())),
                            preferred_element_type=jnp.float32)
        h = (g * lax.logistic(g) * u).astype(jnp.bfloat16)
        o = lax.dot_general(h, wd_ref[0].astype(jnp.bfloat16),
                            (((1,), (1,)), ((), ())),
                            preferred_element_type=jnp.float32)
        out_ref[...] += o


NW_C = 32
PPW_C = NPAIR // NW_C
GCH_C = 64


def _combine_body(ys_hbm, inv_hbm, yu_hbm, inv_v, buf_v, sem):
    wid = lax.axis_index("s") * 2 + lax.axis_index("c")
    base = wid * PPW_C
    pltpu.sync_copy(inv_hbm.at[pl.ds(base, PPW_C)], inv_v)
    for g in range(PPW_C // GCH_C):
        pltpu.async_copy(ys_hbm.at[inv_v.at[pl.ds(g * GCH_C, GCH_C)]],
                         buf_v, sem).wait()
        pltpu.sync_copy(buf_v, yu_hbm.at[pl.ds(base + g * GCH_C, GCH_C)])


def _shared_body(x_ref, wg_ref, wu_ref, wd_ref, yu_ref, wk_ref, out_ref):
    f = pl.program_id(1)

    @pl.when(f == 0)
    def _():
        w = wk_ref[0]
        iota2 = lax.broadcasted_iota(jnp.int32, w.shape, 1)
        w1 = jnp.sum(jnp.where(iota2 == 0, w, 0.0), axis=1, keepdims=True)
        w2 = jnp.sum(jnp.where(iota2 == 1, w, 0.0), axis=1, keepdims=True)
        out_ref[...] = (yu_ref[:, :EMBED_DIM] * w1 +
                        yu_ref[:, EMBED_DIM:] * w2)

    x = x_ref[...]
    g = lax.dot_general(x, wg_ref[...].astype(jnp.bfloat16),
                        (((1,), (1,)), ((), ())),
                        preferred_element_type=jnp.float32)
    u = lax.dot_general(x, wu_ref[...].astype(jnp.bfloat16),
                        (((1,), (1,)), ((), ())),
                        preferred_element_type=jnp.float32)
    h = (g * lax.logistic(g) * u).astype(jnp.bfloat16)
    out_ref[...] += lax.dot_general(h, wd_ref[...].astype(jnp.bfloat16),
                                    (((1,), (1,)), ((), ())),
                                    preferred_element_type=jnp.float32)


def _dispatch_call(eid, x, hist):
    d = x.shape[1]
    mesh_d = plsc.VectorSubcoreMesh(core_axis_name="c", subcore_axis_name="s")
    return pl.kernel(
        _dispatch_body,
        out_type=[
            jax.ShapeDtypeStruct((NROWS, d), jnp.float32),
            jax.ShapeDtypeStruct((NPAIR,), jnp.int32),
            jax.ShapeDtypeStruct((2 * L,), jnp.int32),
        ],
        mesh=mesh_d,
        compiler_params=pltpu.CompilerParams(needs_layout_passes=False),
        scratch_types=[
            pltpu.VMEM((PPW,), jnp.int32),
            pltpu.VMEM((PPW,), jnp.int32),
            pltpu.VMEM((NCH, GCH), jnp.int32),
            pltpu.VMEM((PPW,), jnp.int32),
            pltpu.VMEM((NW_D, L), jnp.int32),
            pltpu.VMEM((2 * L,), jnp.int32),
            pltpu.VMEM((NCH, GCH, d), jnp.float32),
            pltpu.SemaphoreType.DMA,
            pltpu.SemaphoreType.DMA,
            pltpu.SemaphoreType.DMA,
        ],
    )(eid, x, hist)


def _combine_call(ys, inv):
    d = ys.shape[1]
    mesh_c = plsc.VectorSubcoreMesh(core_axis_name="c", subcore_axis_name="s")
    return pl.kernel(
        _combine_body,
        out_type=jax.ShapeDtypeStruct((NPAIR, d), jnp.float32),
        mesh=mesh_c,
        compiler_params=pltpu.CompilerParams(needs_layout_passes=False),
        scratch_types=[
            pltpu.VMEM((PPW_C,), jnp.int32),
            pltpu.VMEM((GCH_C, d), jnp.float32),
            pltpu.SemaphoreType.DMA,
        ],
    )(ys, inv)


def _router_call(x, gate_w, interpret=False):
    d = x.shape[1]
    return pl.pallas_call(
        _router_body,
        grid=(N_TOK_TILES,),
        in_specs=[
            pl.BlockSpec((TOK_TILE, d), lambda t: (t, 0)),
            pl.BlockSpec((NUM_EXPERTS, d), lambda t: (0, 0)),
        ],
        out_specs=[
            pl.BlockSpec((1, TOK_TILE, 2), lambda t: (t, 0, 0)),
            pl.BlockSpec((1, TOK_TILE, 2), lambda t: (t, 0, 0)),
            pl.BlockSpec((1, 4, L), lambda t: (t, 0, 0)),
            pl.BlockSpec((TOK_TILE, d), lambda t: (t, 0)),
        ],
        out_shape=[
            jax.ShapeDtypeStruct((N_TOK_TILES, TOK_TILE, 2), jnp.int32),
            jax.ShapeDtypeStruct((N_TOK_TILES, TOK_TILE, 2), jnp.float32),
            jax.ShapeDtypeStruct((N_TOK_TILES, 4, L), jnp.int32),
            jax.ShapeDtypeStruct((SEQ, d), jnp.bfloat16),
        ],
        interpret=interpret,
    )(x, gate_w)


def _gmm_call(meta, xs, Wg, Wu, Wd, interpret=False):
    d = xs.shape[1]

    def _tc(t, m):
        return jnp.minimum(t, m[24] - 1)

    def _fc(t, f, m):
        return jnp.where(t < m[24], (f + t) % N_FF,
                         (m[24] - 1 + N_FF - 1) % N_FF)

    return pl.pallas_call(
        _gmm_body,
        grid_spec=pltpu.PrefetchScalarGridSpec(
            num_scalar_prefetch=1,
            grid=(NTILES, N_FF),
            in_specs=[
                pl.BlockSpec((ROW_TILE, d), lambda t, f, m: (_tc(t, m), 0)),
                pl.BlockSpec((1, FF_TILE, d),
                             lambda t, f, m: (m[_tc(t, m)], _fc(t, f, m), 0)),
                pl.BlockSpec((1, FF_TILE, d),
                             lambda t, f, m: (m[_tc(t, m)], _fc(t, f, m), 0)),
                pl.BlockSpec((1, d, FF_TILE),
                             lambda t, f, m: (m[_tc(t, m)], 0, _fc(t, f, m))),
            ],
            out_specs=pl.BlockSpec((ROW_TILE, d),
                                   lambda t, f, m: (_tc(t, m), 0)),
        ),
        out_shape=jax.ShapeDtypeStruct((NROWS, d), jnp.float32),
        interpret=interpret,
    )(meta, xs, Wg, Wu, Wd)


def _shared_call(xb, Wg_sh, Wu_sh, Wd_sh, yu2, wk3, interpret=False):
    d = xb.shape[1]
    return pl.pallas_call(
        _shared_body,
        grid=(N_TOK_TILES, N_FF_SH),
        in_specs=[
            pl.BlockSpec((TOK_TILE, d), lambda t, f: (t, 0)),
            pl.BlockSpec((FF_TILE, d), lambda t, f: (f, 0)),
            pl.BlockSpec((FF_TILE, d), lambda t, f: (f, 0)),
            pl.BlockSpec((d, FF_TILE), lambda t, f: (0, f)),
            pl.BlockSpec((TOK_TILE, 2 * d), lambda t, f: (t, 0)),
            pl.BlockSpec((1, TOK_TILE, 2), lambda t, f: (t, 0, 0)),
        ],
        out_specs=pl.BlockSpec((TOK_TILE, d), lambda t, f: (t, 0)),
        out_shape=jax.ShapeDtypeStruct((xb.shape[0], d), jnp.float32),
        interpret=interpret,
    )(xb, Wg_sh, Wu_sh, Wd_sh, yu2, wk3)


@jax.jit
def kernel(hidden_states, gate_w, Wg, Wu, Wd, Wg_sh, Wu_sh, Wd_sh):
    b, s, d = hidden_states.shape
    x = hidden_states.reshape(s, d)

    eid3, wk3, hist3, xb = _router_call(x, gate_w)
    eid = eid3.reshape(NPAIR)
    hist = hist3.reshape(NW_D, L)

    xs, inv, meta = _dispatch_call(eid, x, hist)
    ys = _gmm_call(meta, xs, Wg, Wu, Wd)
    yu = _combine_call(ys, inv)
    y = _shared_call(xb, Wg_sh, Wu_sh, Wd_sh, yu.reshape(SEQ, 2 * d), wk3)

    return y.reshape(b, s, d)


# speedup vs baseline: 2.6903x; 1.0052x over previous
import functools

import jax
import jax.numpy as jnp
from jax import lax
from jax.experimental import pallas as pl
from jax.experimental.pallas import tpu as pltpu
from jax.experimental.pallas import tpu_sc as plsc

EMBED_DIM = 768
NUM_EXPERTS = 8
D_FF = 3072
D_FF_SH = 1536
SEQ = 2048
NPAIR = 2 * SEQ
ROW_TILE = 512
NTILES = 16
NROWS = NTILES * ROW_TILE
FF_TILE = 1536
N_FF = D_FF // FF_TILE
N_FF_SH = D_FF_SH // FF_TILE
TOK_TILE = 256
N_TOK_TILES = SEQ // TOK_TILE
L = 16


def _router_body(x_ref, gw_ref, eid_ref, wk_ref, hist_ref, xb_ref):
    x = x_ref[...]
    xb_ref[...] = x.astype(jnp.bfloat16)
    logits = lax.dot_general(x, gw_ref[...], (((1,), (1,)), ((), ())),
                             preferred_element_type=jnp.float32)
    m = jnp.max(logits, axis=-1, keepdims=True)
    e = jnp.exp(logits - m)
    scores = e / jnp.sum(e, axis=-1, keepdims=True)
    iota = lax.broadcasted_iota(jnp.int32, scores.shape, 1)
    big = jnp.int32(NUM_EXPERTS)
    m1 = jnp.max(scores, axis=-1, keepdims=True)
    i1 = jnp.min(jnp.where(scores == m1, iota, big), axis=-1, keepdims=True)
    excl = iota == i1
    masked = jnp.where(excl, -jnp.inf, scores)
    m2 = jnp.max(masked, axis=-1, keepdims=True)
    i2 = jnp.min(jnp.where((scores == m2) & (~excl), iota, big),
                 axis=-1, keepdims=True)
    w1 = jnp.sum(jnp.where(iota == i1, scores, 0.0), axis=-1, keepdims=True)
    w2 = jnp.sum(jnp.where((iota == i2) & (~excl), scores, 0.0),
                 axis=-1, keepdims=True)
    eid_ref[0] = jnp.concatenate([i1, i2], axis=1)
    wk_ref[0] = jnp.concatenate([w1, w2], axis=1)
    lane = lax.broadcasted_iota(jnp.int32, (TOK_TILE, L), 1)
    ind = ((i1 == lane).astype(jnp.int32) + (i2 == lane).astype(jnp.int32))
    row = lax.broadcasted_iota(jnp.int32, (TOK_TILE, L), 0)
    for h in range(4):
        msk = (row >= h * 64) & (row < (h + 1) * 64)
        hist_ref[0, h, :] = jnp.sum(jnp.where(msk, ind, 0), axis=0)


NW_H = 16
NW_D = 32
PPW = NPAIR // NW_D
NV = PPW // L
GCH = 128
NCH = PPW // GCH


def _lane(v, e):
    iota = lax.iota(jnp.int32, L)
    return jnp.sum(jnp.where(iota == e, v, 0))


def _dispatch_body(eid_hbm, x_hbm, hist_hbm, xs_hbm, inv_hbm, meta_hbm,
                   eid_v, dst1_v, dst2_v, tok_v, tbl_v, meta_v, xbuf_v,
                   semg0, semg1, sems):
    wid = lax.axis_index("s") * 2 + lax.axis_index("c")
    base_p = wid * PPW
    iota = lax.iota(jnp.int32, L)

    pltpu.sync_copy(eid_hbm.at[pl.ds(base_p, PPW)], eid_v)
    pltpu.sync_copy(hist_hbm, tbl_v)

    tot = jnp.zeros((L,), jnp.int32)
    pre = jnp.zeros((L,), jnp.int32)
    for i in range(NW_D):
        row = tbl_v[i]
        tot = tot + row
        pre = pre + jnp.where(jnp.int32(i) < wid, row, 0)
    padded = (tot + ROW_TILE - 1) - ((tot + ROW_TILE - 1) &
                                     jnp.int32(ROW_TILE - 1))
    cum = plsc.cumsum(padded)
    base = cum - padded
    cumt = lax.shift_right_logical(cum, 9)

    @pl.when(wid == 0)
    def _():
        nact = _lane(cumt, NUM_EXPERTS - 1)
        for half in range(2):
            t = iota + half * L
            ex = jnp.zeros((L,), jnp.int32)
            for e in range(NUM_EXPERTS):
                ce = _lane(cumt, e)
                ex = ex + jnp.where(ce <= t, 1, 0)
            ex = jnp.minimum(ex, NUM_EXPERTS - 1)
            if half == 1:
                ex = jnp.where(iota == 8, nact, jnp.where(iota < 8, ex, 0))
            meta_v[pl.ds(half * L, L)] = ex
        pltpu.sync_copy(meta_v, meta_hbm)

    start = base + pre
    for v in range(NV):
        ids = eid_v[pl.ds(v * L, L)]
        dst = jnp.zeros((L,), jnp.int32)
        add = jnp.zeros((L,), jnp.int32)
        for e in range(NUM_EXPERTS):
            msk = ids == e
            rank = plsc.cumsum(jnp.where(msk, 1, 0))
            c = _lane(rank, L - 1)
            se = _lane(start, e)
            dst = jnp.where(msk, se + rank - 1, dst)
            add = add + jnp.where(iota == e, c, 0)
        start = start + add
        dst1_v[pl.ds(v * L, L)] = dst
        dst2_v[v // (GCH // L), pl.ds((v % (GCH // L)) * L, L)] = dst
        tok_v[pl.ds(v * L, L)] = lax.shift_right_logical(
            base_p + v * L + iota, 1)

    pltpu.sync_copy(dst1_v, inv_hbm.at[pl.ds(base_p, PPW)])

    pltpu.async_copy(x_hbm.at[tok_v], xbuf_v.at[0], semg0).wait()
    pltpu.async_copy(xbuf_v.at[0], xs_hbm.at[dst2_v.at[0]], sems).wait()


def _gmm_body(meta_ref, xs_ref, wg_ref, wu_ref, wd_ref, out_ref):
    t = pl.program_id(0)
    f = pl.program_id(1)
    active = t < meta_ref[24]

    @pl.when(active)
    def _():
        @pl.when(f == 0)
        def _():
            out_ref[...] = jnp.zeros_like(out_ref)

        x = xs_ref[...].astype(jnp.bfloat16)
        g = lax.dot_general(x, wg_ref[0].astype(jnp.bfloat16),
                            (((1,), (1,)), ((), ())),
                            preferred_element_type=jnp.float32)
        u = lax.dot_general(x, wu_ref[0].astype(jnp.bfloat16),
                            (((1,), (1,)), ((), ())),
                            preferred_element_type=jnp.float32)
        h = (g * lax.logistic(g) * u).astype(jnp.bfloat16)
        o = lax.dot_general(h, wd_ref[0].astype(jnp.bfloat16),
                            (((1,), (1,)), ((), ())),
                            preferred_element_type=jnp.float32)
        out_ref[...] += o


NW_C = 32
PPW_C = NPAIR // NW_C
GCH_C = 128


def _combine_body(ys_hbm, inv_hbm, yu_hbm, inv_v, buf_v, sem):
    wid = lax.axis_index("s") * 2 + lax.axis_index("c")
    base = wid * PPW_C
    pltpu.sync_copy(inv_hbm.at[pl.ds(base, PPW_C)], inv_v)
    for g in range(PPW_C // GCH_C):
        pltpu.async_copy(ys_hbm.at[inv_v.at[pl.ds(g * GCH_C, GCH_C)]],
                         buf_v, sem).wait()
        pltpu.sync_copy(buf_v, yu_hbm.at[pl.ds(base + g * GCH_C, GCH_C)])


def _shared_body(x_ref, wg_ref, wu_ref, wd_ref, yu_ref, wk_ref, out_ref):
    f = pl.program_id(1)

    @pl.when(f == 0)
    def _():
        w = wk_ref[0]
        iota2 = lax.broadcasted_iota(jnp.int32, w.shape, 1)
        w1 = jnp.sum(jnp.where(iota2 == 0, w, 0.0), axis=1, keepdims=True)
        w2 = jnp.sum(jnp.where(iota2 == 1, w, 0.0), axis=1, keepdims=True)
        out_ref[...] = (yu_ref[:, :EMBED_DIM] * w1 +
                        yu_ref[:, EMBED_DIM:] * w2)

    x = x_ref[...]
    g = lax.dot_general(x, wg_ref[...].astype(jnp.bfloat16),
                        (((1,), (1,)), ((), ())),
                        preferred_element_type=jnp.float32)
    u = lax.dot_general(x, wu_ref[...].astype(jnp.bfloat16),
                        (((1,), (1,)), ((), ())),
                        preferred_element_type=jnp.float32)
    h = (g * lax.logistic(g) * u).astype(jnp.bfloat16)
    out_ref[...] += lax.dot_general(h, wd_ref[...].astype(jnp.bfloat16),
                                    (((1,), (1,)), ((), ())),
                                    preferred_element_type=jnp.float32)


def _dispatch_call(eid, x, hist):
    d = x.shape[1]
    mesh_d = plsc.VectorSubcoreMesh(core_axis_name="c", subcore_axis_name="s")
    return pl.kernel(
        _dispatch_body,
        out_type=[
            jax.ShapeDtypeStruct((NROWS, d), jnp.float32),
            jax.ShapeDtypeStruct((NPAIR,), jnp.int32),
            jax.ShapeDtypeStruct((2 * L,), jnp.int32),
        ],
        mesh=mesh_d,
        compiler_params=pltpu.CompilerParams(needs_layout_passes=False),
        scratch_types=[
            pltpu.VMEM((PPW,), jnp.int32),
            pltpu.VMEM((PPW,), jnp.int32),
            pltpu.VMEM((NCH, GCH), jnp.int32),
            pltpu.VMEM((PPW,), jnp.int32),
            pltpu.VMEM((NW_D, L), jnp.int32),
            pltpu.VMEM((2 * L,), jnp.int32),
            pltpu.VMEM((NCH, GCH, d), jnp.float32),
            pltpu.SemaphoreType.DMA,
            pltpu.SemaphoreType.DMA,
            pltpu.SemaphoreType.DMA,
        ],
    )(eid, x, hist)


def _combine_call(ys, inv):
    d = ys.shape[1]
    mesh_c = plsc.VectorSubcoreMesh(core_axis_name="c", subcore_axis_name="s")
    return pl.kernel(
        _combine_body,
        out_type=jax.ShapeDtypeStruct((NPAIR, d), jnp.float32),
        mesh=mesh_c,
        compiler_params=pltpu.CompilerParams(needs_layout_passes=False),
        scratch_types=[
            pltpu.VMEM((PPW_C,), jnp.int32),
            pltpu.VMEM((GCH_C, d), jnp.float32),
            pltpu.SemaphoreType.DMA,
        ],
    )(ys, inv)


def _router_call(x, gate_w, interpret=False):
    d = x.shape[1]
    return pl.pallas_call(
        _router_body,
        grid=(N_TOK_TILES,),
        in_specs=[
            pl.BlockSpec((TOK_TILE, d), lambda t: (t, 0)),
            pl.BlockSpec((NUM_EXPERTS, d), lambda t: (0, 0)),
        ],
        out_specs=[
            pl.BlockSpec((1, TOK_TILE, 2), lambda t: (t, 0, 0)),
            pl.BlockSpec((1, TOK_TILE, 2), lambda t: (t, 0, 0)),
            pl.BlockSpec((1, 4, L), lambda t: (t, 0, 0)),
            pl.BlockSpec((TOK_TILE, d), lambda t: (t, 0)),
        ],
        out_shape=[
            jax.ShapeDtypeStruct((N_TOK_TILES, TOK_TILE, 2), jnp.int32),
            jax.ShapeDtypeStruct((N_TOK_TILES, TOK_TILE, 2), jnp.float32),
            jax.ShapeDtypeStruct((N_TOK_TILES, 4, L), jnp.int32),
            jax.ShapeDtypeStruct((SEQ, d), jnp.bfloat16),
        ],
        interpret=interpret,
    )(x, gate_w)


def _gmm_call(meta, xs, Wg, Wu, Wd, interpret=False):
    d = xs.shape[1]

    def _tc(t, m):
        return jnp.minimum(t, m[24] - 1)

    def _fc(t, f, m):
        return jnp.where(t < m[24], (f + t) % N_FF,
                         (m[24] - 1 + N_FF - 1) % N_FF)

    return pl.pallas_call(
        _gmm_body,
        grid_spec=pltpu.PrefetchScalarGridSpec(
            num_scalar_prefetch=1,
            grid=(NTILES, N_FF),
            in_specs=[
                pl.BlockSpec((ROW_TILE, d), lambda t, f, m: (_tc(t, m), 0)),
                pl.BlockSpec((1, FF_TILE, d),
                             lambda t, f, m: (m[_tc(t, m)], _fc(t, f, m), 0)),
                pl.BlockSpec((1, FF_TILE, d),
                             lambda t, f, m: (m[_tc(t, m)], _fc(t, f, m), 0)),
                pl.BlockSpec((1, d, FF_TILE),
                             lambda t, f, m: (m[_tc(t, m)], 0, _fc(t, f, m))),
            ],
            out_specs=pl.BlockSpec((ROW_TILE, d),
                                   lambda t, f, m: (_tc(t, m), 0)),
        ),
        out_shape=jax.ShapeDtypeStruct((NROWS, d), jnp.float32),
        interpret=interpret,
    )(meta, xs, Wg, Wu, Wd)


def _shared_call(xb, Wg_sh, Wu_sh, Wd_sh, yu2, wk3, interpret=False):
    d = xb.shape[1]
    return pl.pallas_call(
        _shared_body,
        grid=(N_TOK_TILES, N_FF_SH),
        in_specs=[
            pl.BlockSpec((TOK_TILE, d), lambda t, f: (t, 0)),
            pl.BlockSpec((FF_TILE, d), lambda t, f: (f, 0)),
            pl.BlockSpec((FF_TILE, d), lambda t, f: (f, 0)),
            pl.BlockSpec((d, FF_TILE), lambda t, f: (0, f)),
            pl.BlockSpec((TOK_TILE, 2 * d), lambda t, f: (t, 0)),
            pl.BlockSpec((1, TOK_TILE, 2), lambda t, f: (t, 0, 0)),
        ],
        out_specs=pl.BlockSpec((TOK_TILE, d), lambda t, f: (t, 0)),
        out_shape=jax.ShapeDtypeStruct((xb.shape[0], d), jnp.float32),
        interpret=interpret,
    )(xb, Wg_sh, Wu_sh, Wd_sh, yu2, wk3)


@jax.jit
def kernel(hidden_states, gate_w, Wg, Wu, Wd, Wg_sh, Wu_sh, Wd_sh):
    b, s, d = hidden_states.shape
    x = hidden_states.reshape(s, d)

    eid3, wk3, hist3, xb = _router_call(x, gate_w)
    eid = eid3.reshape(NPAIR)
    hist = hist3.reshape(NW_D, L)

    xs, inv, meta = _dispatch_call(eid, x, hist)
    ys = _gmm_call(meta, xs, Wg, Wu, Wd)
    yu = _combine_call(ys, inv)
    y = _shared_call(xb, Wg_sh, Wu_sh, Wd_sh, yu.reshape(SEQ, 2 * d), wk3)

    return y.reshape(b, s, d)
